# baseline (speedup 1.0000x reference)
import functools

import jax
import jax.numpy as jnp
from jax import lax
from jax.experimental import pallas as pl
from jax.experimental.pallas import tpu as pltpu
from jax.experimental.pallas import tpu_sc as plsc

N = 10000
E = 320000
D = 128
H = 128

_EDGE_BLOCK = 2048

_NUM_WORKERS = 32
_EPW = E // _NUM_WORKERS
_LANES = 16


def _node_proj_body(x_ref, w_ref, b_ref, w1r_ref, w2r_ref, o1_ref, o2_ref):
    x = x_ref[...].astype(jnp.bfloat16)
    n = jnp.dot(x, w_ref[...], preferred_element_type=jnp.float32)
    n = jnp.maximum(n + b_ref[...], 0.0)
    o1_ref[...] = jnp.sum(n * w1r_ref[...], axis=1)
    o2_ref[...] = jnp.sum(n * w2r_ref[...], axis=1)


def _edge_proj_body(x_ref, w_ref, b_ref, w3r_ref, bp_ref, o_ref):
    x = x_ref[...].astype(jnp.bfloat16)
    e = jnp.dot(x, w_ref[...], preferred_element_type=jnp.float32)
    e = jnp.maximum(e + b_ref[...], 0.0)
    o_ref[...] = jnp.sum(e * w3r_ref[...], axis=1) + bp_ref[...]


def _combine_body(p1_hbm, p2_hbm, src_hbm, dst_hbm, pe_hbm, out_hbm,
                  tab1_v, tab2_v, src_v, dst_v, pe_v, out_v):
    wid = lax.axis_index("s") * 2 + lax.axis_index("c")
    base = wid * _EPW
    pltpu.sync_copy(p1_hbm, tab1_v)
    pltpu.sync_copy(p2_hbm, tab2_v)
    pltpu.sync_copy(src_hbm.at[pl.ds(base, _EPW)], src_v)
    pltpu.sync_copy(dst_hbm.at[pl.ds(base, _EPW)], dst_v)
    pltpu.sync_copy(pe_hbm.at[pl.ds(base, _EPW)], pe_v)

    def body(i, carry):
        o = i * _LANES
        s = src_v[pl.ds(o, _LANES)]
        d = dst_v[pl.ds(o, _LANES)]
        a = plsc.load_gather(tab1_v, [s])
        b = plsc.load_gather(tab2_v, [d])
        out_v[pl.ds(o, _LANES)] = a + b + pe_v[pl.ds(o, _LANES)]
        return carry

    lax.fori_loop(0, _EPW // _LANES, body, 0)
    pltpu.sync_copy(out_v, out_hbm.at[pl.ds(base, _EPW)])


def kernel(node_features, edge_features, edge_index, W_node, b_node,
           W_edge, b_edge, W_pred, b_pred):
    w1r = W_pred[0:H].reshape(1, H)
    w2r = W_pred[H:2 * H].reshape(1, H)
    w3r = W_pred[2 * H:3 * H].reshape(1, H)

    p1, p2 = pl.pallas_call(
        _node_proj_body,
        out_shape=(jax.ShapeDtypeStruct((N,), jnp.float32),
                   jax.ShapeDtypeStruct((N,), jnp.float32)),
    )(node_features, W_node.astype(jnp.bfloat16), b_node.reshape(1, H),
      w1r, w2r)

    pe = pl.pallas_call(
        _edge_proj_body,
        grid=(pl.cdiv(E, _EDGE_BLOCK),),
        in_specs=[
            pl.BlockSpec((_EDGE_BLOCK, D), lambda i: (i, 0)),
            pl.BlockSpec((D, H), lambda i: (0, 0)),
            pl.BlockSpec((1, H), lambda i: (0, 0)),
            pl.BlockSpec((1, H), lambda i: (0, 0)),
            pl.BlockSpec((1,), lambda i: (0,)),
        ],
        out_specs=pl.BlockSpec((_EDGE_BLOCK,), lambda i: (i,)),
        out_shape=jax.ShapeDtypeStruct((E,), jnp.float32),
    )(edge_features, W_edge.astype(jnp.bfloat16), b_edge.reshape(1, H),
      w3r, b_pred)

    combine = functools.partial(
        pl.kernel,
        out_type=jax.ShapeDtypeStruct((E,), jnp.float32),
        mesh=plsc.VectorSubcoreMesh(core_axis_name="c", subcore_axis_name="s"),
        compiler_params=pltpu.CompilerParams(needs_layout_passes=False),
        scratch_types=[
            pltpu.VMEM((N,), jnp.float32),
            pltpu.VMEM((N,), jnp.float32),
            pltpu.VMEM((_EPW,), jnp.int32),
            pltpu.VMEM((_EPW,), jnp.int32),
            pltpu.VMEM((_EPW,), jnp.float32),
            pltpu.VMEM((_EPW,), jnp.float32),
        ],
    )(_combine_body)

    out = combine(p1, p2, edge_index[0], edge_index[1], pe)
    return out.reshape(E, 1)


# speedup vs baseline: 1.4196x; 1.4196x over previous
import functools

import jax
import jax.numpy as jnp
from jax import lax
from jax.experimental import pallas as pl
from jax.experimental.pallas import tpu as pltpu
from jax.experimental.pallas import tpu_sc as plsc

N = 10000
E = 320000
D = 128
H = 128

_EDGE_BLOCK = 2048

_NUM_WORKERS = 32
_EPW = E // _NUM_WORKERS
_LANES = 16


def _node_proj_body(x_ref, w_ref, b_ref, w1c_ref, w2c_ref, o1_ref, o2_ref):
    x = x_ref[...].astype(jnp.bfloat16)
    n = jnp.dot(x, w_ref[...], preferred_element_type=jnp.float32)
    n = jnp.maximum(n + b_ref[...], 0.0)
    nt = n.T
    o1_ref[...] = jnp.sum(nt * w1c_ref[...], axis=0)
    o2_ref[...] = jnp.sum(nt * w2c_ref[...], axis=0)


def _edge_proj_body(x_ref, w_ref, b_ref, w3c_ref, bp_ref, o_ref):
    x = x_ref[...].astype(jnp.bfloat16)
    e = jnp.dot(x, w_ref[...], preferred_element_type=jnp.float32)
    e = jnp.maximum(e + b_ref[...], 0.0)
    o_ref[...] = jnp.sum(e.T * w3c_ref[...], axis=0) + bp_ref[...]


def _combine_body(p1_hbm, p2_hbm, src_hbm, dst_hbm, pe_hbm, out_hbm,
                  tab1_v, tab2_v, src_v, dst_v, pe_v, out_v):
    wid = lax.axis_index("s") * 2 + lax.axis_index("c")
    base = wid * _EPW
    pltpu.sync_copy(p1_hbm, tab1_v)
    pltpu.sync_copy(p2_hbm, tab2_v)
    pltpu.sync_copy(src_hbm.at[pl.ds(base, _EPW)], src_v)
    pltpu.sync_copy(dst_hbm.at[pl.ds(base, _EPW)], dst_v)
    pltpu.sync_copy(pe_hbm.at[pl.ds(base, _EPW)], pe_v)

    def body(i, carry):
        o = i * _LANES
        s = src_v[pl.ds(o, _LANES)]
        d = dst_v[pl.ds(o, _LANES)]
        a = plsc.load_gather(tab1_v, [s])
        b = plsc.load_gather(tab2_v, [d])
        out_v[pl.ds(o, _LANES)] = a + b + pe_v[pl.ds(o, _LANES)]
        return carry

    lax.fori_loop(0, _EPW // _LANES, body, 0)
    pltpu.sync_copy(out_v, out_hbm.at[pl.ds(base, _EPW)])


def kernel(node_features, edge_features, edge_index, W_node, b_node,
           W_edge, b_edge, W_pred, b_pred):
    w1c = W_pred[0:H]
    w2c = W_pred[H:2 * H]
    w3c = W_pred[2 * H:3 * H]

    p1, p2 = pl.pallas_call(
        _node_proj_body,
        out_shape=(jax.ShapeDtypeStruct((N,), jnp.float32),
                   jax.ShapeDtypeStruct((N,), jnp.float32)),
    )(node_features, W_node.astype(jnp.bfloat16), b_node.reshape(1, H),
      w1c, w2c)

    pe = pl.pallas_call(
        _edge_proj_body,
        grid=(pl.cdiv(E, _EDGE_BLOCK),),
        in_specs=[
            pl.BlockSpec((_EDGE_BLOCK, D), lambda i: (i, 0)),
            pl.BlockSpec((D, H), lambda i: (0, 0)),
            pl.BlockSpec((1, H), lambda i: (0, 0)),
            pl.BlockSpec((H, 1), lambda i: (0, 0)),
            pl.BlockSpec((1,), lambda i: (0,)),
        ],
        out_specs=pl.BlockSpec((_EDGE_BLOCK,), lambda i: (i,)),
        out_shape=jax.ShapeDtypeStruct((E,), jnp.float32),
    )(edge_features, W_edge.astype(jnp.bfloat16), b_edge.reshape(1, H),
      w3c, b_pred)

    combine = functools.partial(
        pl.kernel,
        out_type=jax.ShapeDtypeStruct((E,), jnp.float32),
        mesh=plsc.VectorSubcoreMesh(core_axis_name="c", subcore_axis_name="s"),
        compiler_params=pltpu.CompilerParams(needs_layout_passes=False),
        scratch_types=[
            pltpu.VMEM((N,), jnp.float32),
            pltpu.VMEM((N,), jnp.float32),
            pltpu.VMEM((_EPW,), jnp.int32),
            pltpu.VMEM((_EPW,), jnp.int32),
            pltpu.VMEM((_EPW,), jnp.float32),
            pltpu.VMEM((_EPW,), jnp.float32),
        ],
    )(_combine_body)

    out = combine(p1, p2, edge_index[0], edge_index[1], pe)
    return out.reshape(E, 1)


# speedup vs baseline: 1.8492x; 1.3026x over previous
import functools

import jax
import jax.numpy as jnp
from jax import lax
from jax.experimental import pallas as pl
from jax.experimental.pallas import tpu as pltpu
from jax.experimental.pallas import tpu_sc as plsc

N = 10000
E = 320000
D = 128
H = 128

_EDGE_BLOCK = 4096

_NUM_WORKERS = 32
_EPW = E // _NUM_WORKERS
_LANES = 16


def _node_proj_body(x_ref, w_ref, b_ref, w1c_ref, w2c_ref, o1_ref, o2_ref):
    x = x_ref[...].astype(jnp.bfloat16)
    w = w_ref[...].astype(jnp.bfloat16)
    n = jnp.dot(x, w, preferred_element_type=jnp.float32)
    n = jnp.maximum(n + b_ref[...], 0.0)
    nt = n.T
    o1_ref[...] = jnp.sum(nt * w1c_ref[...], axis=0)
    o2_ref[...] = jnp.sum(nt * w2c_ref[...], axis=0)


def _edge_proj_body(x_ref, w_ref, b_ref, w3c_ref, bp_ref, o_ref):
    x = x_ref[...].astype(jnp.bfloat16)
    w = w_ref[...].astype(jnp.bfloat16)
    e = jnp.dot(x, w, preferred_element_type=jnp.float32)
    e = jnp.maximum(e + b_ref[...], 0.0)
    o_ref[...] = jnp.sum(e.T * w3c_ref[...], axis=0) + bp_ref[...]


def _combine_body(p1_hbm, p2_hbm, src_hbm, dst_hbm, pe_hbm, out_hbm,
                  tab1_v, tab2_v, src_v, dst_v, pe_v, out_v):
    wid = lax.axis_index("s") * 2 + lax.axis_index("c")
    base = wid * _EPW
    pltpu.sync_copy(p1_hbm, tab1_v)
    pltpu.sync_copy(p2_hbm, tab2_v)
    pltpu.sync_copy(src_hbm.at[pl.ds(base, _EPW)], src_v)
    pltpu.sync_copy(dst_hbm.at[pl.ds(base, _EPW)], dst_v)
    pltpu.sync_copy(pe_hbm.at[pl.ds(base, _EPW)], pe_v)

    def body(i, carry):
        o = i * _LANES
        s = src_v[pl.ds(o, _LANES)]
        d = dst_v[pl.ds(o, _LANES)]
        a = plsc.load_gather(tab1_v, [s])
        b = plsc.load_gather(tab2_v, [d])
        out_v[pl.ds(o, _LANES)] = a + b + pe_v[pl.ds(o, _LANES)]
        return carry

    lax.fori_loop(0, _EPW // _LANES, body, 0)
    pltpu.sync_copy(out_v, out_hbm.at[pl.ds(base, _EPW)])


def kernel(node_features, edge_features, edge_index, W_node, b_node,
           W_edge, b_edge, W_pred, b_pred):
    w1c = W_pred[0:H]
    w2c = W_pred[H:2 * H]
    w3c = W_pred[2 * H:3 * H]

    p1, p2 = pl.pallas_call(
        _node_proj_body,
        out_shape=(jax.ShapeDtypeStruct((N,), jnp.float32),
                   jax.ShapeDtypeStruct((N,), jnp.float32)),
    )(node_features, W_node, b_node.reshape(1, H), w1c, w2c)

    pe = pl.pallas_call(
        _edge_proj_body,
        grid=(pl.cdiv(E, _EDGE_BLOCK),),
        in_specs=[
            pl.BlockSpec((_EDGE_BLOCK, D), lambda i: (i, 0)),
            pl.BlockSpec((D, H), lambda i: (0, 0)),
            pl.BlockSpec((1, H), lambda i: (0, 0)),
            pl.BlockSpec((H, 1), lambda i: (0, 0)),
            pl.BlockSpec((1,), lambda i: (0,)),
        ],
        out_specs=pl.BlockSpec((_EDGE_BLOCK,), lambda i: (i,)),
        out_shape=jax.ShapeDtypeStruct((E,), jnp.float32),
    )(edge_features, W_edge, b_edge.reshape(1, H), w3c, b_pred)

    combine = functools.partial(
        pl.kernel,
        out_type=jax.ShapeDtypeStruct((E,), jnp.float32),
        mesh=plsc.VectorSubcoreMesh(core_axis_name="c", subcore_axis_name="s"),
        compiler_params=pltpu.CompilerParams(needs_layout_passes=False),
        scratch_types=[
            pltpu.VMEM((N,), jnp.float32),
            pltpu.VMEM((N,), jnp.float32),
            pltpu.VMEM((_EPW,), jnp.int32),
            pltpu.VMEM((_EPW,), jnp.int32),
            pltpu.VMEM((_EPW,), jnp.float32),
            pltpu.VMEM((_EPW,), jnp.float32),
        ],
    )(_combine_body)

    out = combine(p1, p2, edge_index[0], edge_index[1], pe)
    return out.reshape(E, 1)


# speedup vs baseline: 2.1596x; 1.1678x over previous
import functools

import jax
import jax.numpy as jnp
from jax import lax
from jax.experimental import pallas as pl
from jax.experimental.pallas import tpu as pltpu
from jax.experimental.pallas import tpu_sc as plsc

N = 10000
E = 320000
D = 128
H = 128

_EDGE_BLOCK = 8192

_NUM_WORKERS = 32
_EPW = E // _NUM_WORKERS
_LANES = 16


def _node_proj_body(x_ref, w_ref, b_ref, w1c_ref, w2c_ref, o1_ref, o2_ref):
    x = x_ref[...].astype(jnp.bfloat16)
    w = w_ref[...].astype(jnp.bfloat16)
    n = jnp.dot(x, w, preferred_element_type=jnp.float32)
    n = jnp.maximum(n + b_ref[...], 0.0)
    nt = n.T
    o1_ref[...] = jnp.sum(nt * w1c_ref[...], axis=0)
    o2_ref[...] = jnp.sum(nt * w2c_ref[...], axis=0)


def _edge_proj_body(x_ref, w_ref, b_ref, w3c_ref, bp_ref, o_ref):
    x = x_ref[...].astype(jnp.bfloat16)
    w = w_ref[...].astype(jnp.bfloat16)
    e = jnp.dot(x, w, preferred_element_type=jnp.float32)
    e = jnp.maximum(e + b_ref[...], 0.0)
    o_ref[...] = jnp.sum(e.T * w3c_ref[...], axis=0) + bp_ref[...]


def _combine_body(p1_hbm, p2_hbm, src_hbm, dst_hbm, pe_hbm, out_hbm,
                  tab1_v, tab2_v, src_v, dst_v, pe_v, out_v):
    wid = lax.axis_index("s") * 2 + lax.axis_index("c")
    base = wid * _EPW
    pltpu.sync_copy(p1_hbm, tab1_v)
    pltpu.sync_copy(p2_hbm, tab2_v)
    pltpu.sync_copy(src_hbm.at[pl.ds(base, _EPW)], src_v)
    pltpu.sync_copy(dst_hbm.at[pl.ds(base, _EPW)], dst_v)
    pltpu.sync_copy(pe_hbm.at[pl.ds(base, _EPW)], pe_v)

    def body(i, carry):
        o = i * _LANES
        s = src_v[pl.ds(o, _LANES)]
        d = dst_v[pl.ds(o, _LANES)]
        a = plsc.load_gather(tab1_v, [s])
        b = plsc.load_gather(tab2_v, [d])
        out_v[pl.ds(o, _LANES)] = a + b + pe_v[pl.ds(o, _LANES)]
        return carry

    lax.fori_loop(0, _EPW // _LANES, body, 0)
    pltpu.sync_copy(out_v, out_hbm.at[pl.ds(base, _EPW)])


def kernel(node_features, edge_features, edge_index, W_node, b_node,
           W_edge, b_edge, W_pred, b_pred):
    w1c = W_pred[0:H]
    w2c = W_pred[H:2 * H]
    w3c = W_pred[2 * H:3 * H]

    p1, p2 = pl.pallas_call(
        _node_proj_body,
        out_shape=(jax.ShapeDtypeStruct((N,), jnp.float32),
                   jax.ShapeDtypeStruct((N,), jnp.float32)),
    )(node_features, W_node, b_node.reshape(1, H), w1c, w2c)

    pe = pl.pallas_call(
        _edge_proj_body,
        grid=(pl.cdiv(E, _EDGE_BLOCK),),
        in_specs=[
            pl.BlockSpec((_EDGE_BLOCK, D), lambda i: (i, 0)),
            pl.BlockSpec((D, H), lambda i: (0, 0)),
            pl.BlockSpec((1, H), lambda i: (0, 0)),
            pl.BlockSpec((H, 1), lambda i: (0, 0)),
            pl.BlockSpec((1,), lambda i: (0,)),
        ],
        out_specs=pl.BlockSpec((_EDGE_BLOCK,), lambda i: (i,)),
        out_shape=jax.ShapeDtypeStruct((E,), jnp.float32),
    )(edge_features, W_edge, b_edge.reshape(1, H), w3c, b_pred)

    combine = functools.partial(
        pl.kernel,
        out_type=jax.ShapeDtypeStruct((E,), jnp.float32),
        mesh=plsc.VectorSubcoreMesh(core_axis_name="c", subcore_axis_name="s"),
        compiler_params=pltpu.CompilerParams(needs_layout_passes=False),
        scratch_types=[
            pltpu.VMEM((N,), jnp.float32),
            pltpu.VMEM((N,), jnp.float32),
            pltpu.VMEM((_EPW,), jnp.int32),
            pltpu.VMEM((_EPW,), jnp.int32),
            pltpu.VMEM((_EPW,), jnp.float32),
            pltpu.VMEM((_EPW,), jnp.float32),
        ],
    )(_combine_body)

    out = combine(p1, p2, edge_index[0], edge_index[1], pe)
    return out.reshape(E, 1)


# speedup vs baseline: 2.3549x; 1.0905x over previous
import functools

import jax
import jax.numpy as jnp
from jax import lax
from jax.experimental import pallas as pl
from jax.experimental.pallas import tpu as pltpu
from jax.experimental.pallas import tpu_sc as plsc

N = 10000
E = 320000
D = 128
H = 128

_EDGE_BLOCK = 16384

_NUM_WORKERS = 32
_EPW = E // _NUM_WORKERS
_LANES = 16


def _node_proj_body(x_ref, w_ref, b_ref, w1c_ref, w2c_ref, o1_ref, o2_ref):
    x = x_ref[...].astype(jnp.bfloat16)
    w = w_ref[...].astype(jnp.bfloat16)
    n = jnp.dot(x, w, preferred_element_type=jnp.float32)
    n = jnp.maximum(n + b_ref[...], 0.0)
    nt = n.T
    o1_ref[...] = jnp.sum(nt * w1c_ref[...], axis=0)
    o2_ref[...] = jnp.sum(nt * w2c_ref[...], axis=0)


def _edge_proj_body(x_ref, w_ref, b_ref, w3c_ref, bp_ref, o_ref):
    x = x_ref[...].astype(jnp.bfloat16)
    w = w_ref[...].astype(jnp.bfloat16)
    e = jnp.dot(x, w, preferred_element_type=jnp.float32)
    e = jnp.maximum(e + b_ref[...], 0.0)
    o_ref[...] = jnp.sum(e.T * w3c_ref[...], axis=0) + bp_ref[...]


def _combine_body(p1_hbm, p2_hbm, src_hbm, dst_hbm, pe_hbm, out_hbm,
                  tab1_v, tab2_v, src_v, dst_v, pe_v, out_v):
    wid = lax.axis_index("s") * 2 + lax.axis_index("c")
    base = wid * _EPW
    pltpu.sync_copy(p1_hbm, tab1_v)
    pltpu.sync_copy(p2_hbm, tab2_v)
    pltpu.sync_copy(src_hbm.at[pl.ds(base, _EPW)], src_v)
    pltpu.sync_copy(dst_hbm.at[pl.ds(base, _EPW)], dst_v)
    pltpu.sync_copy(pe_hbm.at[pl.ds(base, _EPW)], pe_v)

    def body(i, carry):
        o = i * _LANES
        s = src_v[pl.ds(o, _LANES)]
        d = dst_v[pl.ds(o, _LANES)]
        a = plsc.load_gather(tab1_v, [s])
        b = plsc.load_gather(tab2_v, [d])
        out_v[pl.ds(o, _LANES)] = a + b + pe_v[pl.ds(o, _LANES)]
        return carry

    lax.fori_loop(0, _EPW // _LANES, body, 0)
    pltpu.sync_copy(out_v, out_hbm.at[pl.ds(base, _EPW)])


def kernel(node_features, edge_features, edge_index, W_node, b_node,
           W_edge, b_edge, W_pred, b_pred):
    w1c = W_pred[0:H]
    w2c = W_pred[H:2 * H]
    w3c = W_pred[2 * H:3 * H]

    p1, p2 = pl.pallas_call(
        _node_proj_body,
        out_shape=(jax.ShapeDtypeStruct((N,), jnp.float32),
                   jax.ShapeDtypeStruct((N,), jnp.float32)),
    )(node_features, W_node, b_node.reshape(1, H), w1c, w2c)

    pe = pl.pallas_call(
        _edge_proj_body,
        grid=(pl.cdiv(E, _EDGE_BLOCK),),
        in_specs=[
            pl.BlockSpec((_EDGE_BLOCK, D), lambda i: (i, 0)),
            pl.BlockSpec((D, H), lambda i: (0, 0)),
            pl.BlockSpec((1, H), lambda i: (0, 0)),
            pl.BlockSpec((H, 1), lambda i: (0, 0)),
            pl.BlockSpec((1,), lambda i: (0,)),
        ],
        out_specs=pl.BlockSpec((_EDGE_BLOCK,), lambda i: (i,)),
        out_shape=jax.ShapeDtypeStruct((E,), jnp.float32),
    )(edge_features, W_edge, b_edge.reshape(1, H), w3c, b_pred)

    combine = functools.partial(
        pl.kernel,
        out_type=jax.ShapeDtypeStruct((E,), jnp.float32),
        mesh=plsc.VectorSubcoreMesh(core_axis_name="c", subcore_axis_name="s"),
        compiler_params=pltpu.CompilerParams(needs_layout_passes=False),
        scratch_types=[
            pltpu.VMEM((N,), jnp.float32),
            pltpu.VMEM((N,), jnp.float32),
            pltpu.VMEM((_EPW,), jnp.int32),
            pltpu.VMEM((_EPW,), jnp.int32),
            pltpu.VMEM((_EPW,), jnp.float32),
            pltpu.VMEM((_EPW,), jnp.float32),
        ],
    )(_combine_body)

    out = combine(p1, p2, edge_index[0], edge_index[1], pe)
    return out.reshape(E, 1)


# speedup vs baseline: 2.4394x; 1.0359x over previous
import functools

import jax
import jax.numpy as jnp
from jax import lax
from jax.experimental import pallas as pl
from jax.experimental.pallas import tpu as pltpu
from jax.experimental.pallas import tpu_sc as plsc

N = 10000
E = 320000
D = 128
H = 128

_EDGE_BLOCK = 32768

_NUM_WORKERS = 32
_EPW = E // _NUM_WORKERS
_LANES = 16


def _node_proj_body(x_ref, w_ref, b_ref, w1c_ref, w2c_ref, o1_ref, o2_ref):
    x = x_ref[...].astype(jnp.bfloat16)
    w = w_ref[...].astype(jnp.bfloat16)
    n = jnp.dot(x, w, preferred_element_type=jnp.float32)
    n = jnp.maximum(n + b_ref[...], 0.0)
    nt = n.T
    o1_ref[...] = jnp.sum(nt * w1c_ref[...], axis=0)
    o2_ref[...] = jnp.sum(nt * w2c_ref[...], axis=0)


def _edge_proj_body(x_ref, w_ref, b_ref, w3c_ref, bp_ref, o_ref):
    x = x_ref[...].astype(jnp.bfloat16)
    w = w_ref[...].astype(jnp.bfloat16)
    e = jnp.dot(x, w, preferred_element_type=jnp.float32)
    e = jnp.maximum(e + b_ref[...], 0.0)
    o_ref[...] = jnp.sum(e.T * w3c_ref[...], axis=0) + bp_ref[...]


def _combine_body(p1_hbm, p2_hbm, src_hbm, dst_hbm, pe_hbm, out_hbm,
                  tab1_v, tab2_v, src_v, dst_v, pe_v, out_v):
    wid = lax.axis_index("s") * 2 + lax.axis_index("c")
    base = wid * _EPW
    pltpu.sync_copy(p1_hbm, tab1_v)
    pltpu.sync_copy(p2_hbm, tab2_v)
    pltpu.sync_copy(src_hbm.at[pl.ds(base, _EPW)], src_v)
    pltpu.sync_copy(dst_hbm.at[pl.ds(base, _EPW)], dst_v)
    pltpu.sync_copy(pe_hbm.at[pl.ds(base, _EPW)], pe_v)

    def body(i, carry):
        o = i * _LANES
        s = src_v[pl.ds(o, _LANES)]
        d = dst_v[pl.ds(o, _LANES)]
        a = plsc.load_gather(tab1_v, [s])
        b = plsc.load_gather(tab2_v, [d])
        out_v[pl.ds(o, _LANES)] = a + b + pe_v[pl.ds(o, _LANES)]
        return carry

    lax.fori_loop(0, _EPW // _LANES, body, 0)
    pltpu.sync_copy(out_v, out_hbm.at[pl.ds(base, _EPW)])


def kernel(node_features, edge_features, edge_index, W_node, b_node,
           W_edge, b_edge, W_pred, b_pred):
    w1c = W_pred[0:H]
    w2c = W_pred[H:2 * H]
    w3c = W_pred[2 * H:3 * H]

    p1, p2 = pl.pallas_call(
        _node_proj_body,
        out_shape=(jax.ShapeDtypeStruct((N,), jnp.float32),
                   jax.ShapeDtypeStruct((N,), jnp.float32)),
    )(node_features, W_node, b_node.reshape(1, H), w1c, w2c)

    pe = pl.pallas_call(
        _edge_proj_body,
        grid=(pl.cdiv(E, _EDGE_BLOCK),),
        in_specs=[
            pl.BlockSpec((_EDGE_BLOCK, D), lambda i: (i, 0)),
            pl.BlockSpec((D, H), lambda i: (0, 0)),
            pl.BlockSpec((1, H), lambda i: (0, 0)),
            pl.BlockSpec((H, 1), lambda i: (0, 0)),
            pl.BlockSpec((1,), lambda i: (0,)),
        ],
        out_specs=pl.BlockSpec((_EDGE_BLOCK,), lambda i: (i,)),
        out_shape=jax.ShapeDtypeStruct((E,), jnp.float32),
    )(edge_features, W_edge, b_edge.reshape(1, H), w3c, b_pred)

    combine = functools.partial(
        pl.kernel,
        out_type=jax.ShapeDtypeStruct((E,), jnp.float32),
        mesh=plsc.VectorSubcoreMesh(core_axis_name="c", subcore_axis_name="s"),
        compiler_params=pltpu.CompilerParams(needs_layout_passes=False),
        scratch_types=[
            pltpu.VMEM((N,), jnp.float32),
            pltpu.VMEM((N,), jnp.float32),
            pltpu.VMEM((_EPW,), jnp.int32),
            pltpu.VMEM((_EPW,), jnp.int32),
            pltpu.VMEM((_EPW,), jnp.float32),
            pltpu.VMEM((_EPW,), jnp.float32),
        ],
    )(_combine_body)

    out = combine(p1, p2, edge_index[0], edge_index[1], pe)
    return out.reshape(E, 1)


# speedup vs baseline: 2.8136x; 1.1534x over previous
import functools

import jax
import jax.numpy as jnp
from jax import lax
from jax.experimental import pallas as pl
from jax.experimental.pallas import tpu as pltpu
from jax.experimental.pallas import tpu_sc as plsc

N = 10000
E = 320000
D = 128
H = 128

_EDGE_BLOCK = 32768

_NUM_WORKERS = 32
_EPW = E // _NUM_WORKERS
_LANES = 16


def _node_proj_body(x_ref, w_ref, b_ref, w1c_ref, w2c_ref, o1_ref, o2_ref):
    x = x_ref[...].astype(jnp.bfloat16)
    w = w_ref[...].astype(jnp.bfloat16)
    n = jnp.dot(x, w, preferred_element_type=jnp.float32)
    n = jnp.maximum(n + b_ref[...], 0.0)
    nt = n.T
    o1_ref[...] = jnp.sum(nt * w1c_ref[...], axis=0)
    o2_ref[...] = jnp.sum(nt * w2c_ref[...], axis=0)


def _edge_proj_body(x_ref, ei_ref, w_ref, b_ref, w3c_ref, bp_ref,
                    o_ref, src_ref, dst_ref):
    x = x_ref[...].astype(jnp.bfloat16)
    w = w_ref[...].astype(jnp.bfloat16)
    e = jnp.dot(x, w, preferred_element_type=jnp.float32)
    e = jnp.maximum(e + b_ref[...], 0.0)
    o_ref[...] = jnp.sum(e.T * w3c_ref[...], axis=0) + bp_ref[...]
    src_ref[...] = ei_ref[0]
    dst_ref[...] = ei_ref[1]


def _combine_body(p1_hbm, p2_hbm, src_hbm, dst_hbm, pe_hbm, out_hbm,
                  tab1_v, tab2_v, src_v, dst_v, pe_v, out_v, sem):
    wid = lax.axis_index("s") * 2 + lax.axis_index("c")
    base = wid * _EPW
    sl = pl.ds(base, _EPW)
    copies = [
        pltpu.async_copy(p1_hbm, tab1_v, sem),
        pltpu.async_copy(p2_hbm, tab2_v, sem),
        pltpu.async_copy(src_hbm.at[sl], src_v, sem),
        pltpu.async_copy(dst_hbm.at[sl], dst_v, sem),
        pltpu.async_copy(pe_hbm.at[sl], pe_v, sem),
    ]
    for c in copies:
        c.wait()

    _UNROLL = 5

    def body(i, carry):
        for j in range(_UNROLL):
            o = (i * _UNROLL + j) * _LANES
            s = src_v[pl.ds(o, _LANES)]
            d = dst_v[pl.ds(o, _LANES)]
            a = plsc.load_gather(tab1_v, [s])
            b = plsc.load_gather(tab2_v, [d])
            out_v[pl.ds(o, _LANES)] = a + b + pe_v[pl.ds(o, _LANES)]
        return carry

    lax.fori_loop(0, _EPW // (_LANES * _UNROLL), body, 0)
    pltpu.sync_copy(out_v, out_hbm.at[sl])


def kernel(node_features, edge_features, edge_index, W_node, b_node,
           W_edge, b_edge, W_pred, b_pred):
    w1c = W_pred[0:H]
    w2c = W_pred[H:2 * H]
    w3c = W_pred[2 * H:3 * H]

    p1, p2 = pl.pallas_call(
        _node_proj_body,
        out_shape=(jax.ShapeDtypeStruct((N,), jnp.float32),
                   jax.ShapeDtypeStruct((N,), jnp.float32)),
    )(node_features, W_node, b_node.reshape(1, H), w1c, w2c)

    pe, src, dst = pl.pallas_call(
        _edge_proj_body,
        grid=(pl.cdiv(E, _EDGE_BLOCK),),
        in_specs=[
            pl.BlockSpec((_EDGE_BLOCK, D), lambda i: (i, 0)),
            pl.BlockSpec((2, _EDGE_BLOCK), lambda i: (0, i)),
            pl.BlockSpec((D, H), lambda i: (0, 0)),
            pl.BlockSpec((1, H), lambda i: (0, 0)),
            pl.BlockSpec((H, 1), lambda i: (0, 0)),
            pl.BlockSpec((1,), lambda i: (0,)),
        ],
        out_specs=(pl.BlockSpec((_EDGE_BLOCK,), lambda i: (i,)),
                   pl.BlockSpec((_EDGE_BLOCK,), lambda i: (i,)),
                   pl.BlockSpec((_EDGE_BLOCK,), lambda i: (i,))),
        out_shape=(jax.ShapeDtypeStruct((E,), jnp.float32),
                   jax.ShapeDtypeStruct((E,), jnp.int32),
                   jax.ShapeDtypeStruct((E,), jnp.int32)),
    )(edge_features, edge_index, W_edge, b_edge.reshape(1, H), w3c, b_pred)

    combine = functools.partial(
        pl.kernel,
        out_type=jax.ShapeDtypeStruct((E,), jnp.float32),
        mesh=plsc.VectorSubcoreMesh(core_axis_name="c", subcore_axis_name="s"),
        compiler_params=pltpu.CompilerParams(needs_layout_passes=False),
        scratch_types=[
            pltpu.VMEM((N,), jnp.float32),
            pltpu.VMEM((N,), jnp.float32),
            pltpu.VMEM((_EPW,), jnp.int32),
            pltpu.VMEM((_EPW,), jnp.int32),
            pltpu.VMEM((_EPW,), jnp.float32),
            pltpu.VMEM((_EPW,), jnp.float32),
            pltpu.SemaphoreType.DMA,
        ],
    )(_combine_body)

    out = combine(p1, p2, src, dst, pe)
    return out.reshape(E, 1)


# speedup vs baseline: 2.8322x; 1.0066x over previous
import functools

import jax
import jax.numpy as jnp
from jax import lax
from jax.experimental import pallas as pl
from jax.experimental.pallas import tpu as pltpu
from jax.experimental.pallas import tpu_sc as plsc

N = 10000
E = 320000
D = 128
H = 128

_EDGE_BLOCK = 32768

_NUM_WORKERS = 32
_EPW = E // _NUM_WORKERS
_LANES = 16


def _node_proj_body(x_ref, w_ref, b_ref, w1c_ref, w2c_ref, o1_ref, o2_ref):
    x = x_ref[...].astype(jnp.bfloat16)
    w = w_ref[...].astype(jnp.bfloat16)
    n = jnp.dot(x, w, preferred_element_type=jnp.float32)
    n = jnp.maximum(n + b_ref[...], 0.0)
    nt = n.T
    o1_ref[...] = jnp.sum(nt * w1c_ref[...], axis=0)
    o2_ref[...] = jnp.sum(nt * w2c_ref[...], axis=0)


def _edge_proj_body(x_ref, ei_ref, w_ref, b_ref, w3c_ref, bp_ref,
                    o_ref, comb_ref):
    x = x_ref[...].astype(jnp.bfloat16)
    w = w_ref[...].astype(jnp.bfloat16)
    e = jnp.dot(x, w, preferred_element_type=jnp.float32)
    e = jnp.maximum(e + b_ref[...], 0.0)
    o_ref[...] = jnp.sum(e.T * w3c_ref[...], axis=0) + bp_ref[...]
    comb_ref[...] = ei_ref[0] | (ei_ref[1] << 16)


def _combine_body(p1_hbm, p2_hbm, comb_hbm, pe_hbm, out_hbm,
                  tab1_v, tab2_v, comb_v, pe_v, out_v, sem):
    wid = lax.axis_index("s") * 2 + lax.axis_index("c")
    base = wid * _EPW
    sl = pl.ds(base, _EPW)
    copies = [
        pltpu.async_copy(p1_hbm, tab1_v, sem),
        pltpu.async_copy(p2_hbm, tab2_v, sem),
        pltpu.async_copy(comb_hbm.at[sl], comb_v, sem),
        pltpu.async_copy(pe_hbm.at[sl], pe_v, sem),
    ]
    for c in copies:
        c.wait()

    _UNROLL = 5

    def body(i, carry):
        for j in range(_UNROLL):
            o = (i * _UNROLL + j) * _LANES
            c = comb_v[pl.ds(o, _LANES)]
            s = c & 0xFFFF
            d = lax.shift_right_logical(c, 16)
            a = plsc.load_gather(tab1_v, [s])
            b = plsc.load_gather(tab2_v, [d])
            out_v[pl.ds(o, _LANES)] = a + b + pe_v[pl.ds(o, _LANES)]
        return carry

    lax.fori_loop(0, _EPW // (_LANES * _UNROLL), body, 0)
    pltpu.sync_copy(out_v, out_hbm.at[sl])


def kernel(node_features, edge_features, edge_index, W_node, b_node,
           W_edge, b_edge, W_pred, b_pred):
    w1c = W_pred[0:H]
    w2c = W_pred[H:2 * H]
    w3c = W_pred[2 * H:3 * H]

    p1, p2 = pl.pallas_call(
        _node_proj_body,
        out_shape=(jax.ShapeDtypeStruct((N,), jnp.float32),
                   jax.ShapeDtypeStruct((N,), jnp.float32)),
    )(node_features, W_node, b_node.reshape(1, H), w1c, w2c)

    pe, comb = pl.pallas_call(
        _edge_proj_body,
        grid=(pl.cdiv(E, _EDGE_BLOCK),),
        in_specs=[
            pl.BlockSpec((_EDGE_BLOCK, D), lambda i: (i, 0)),
            pl.BlockSpec((2, _EDGE_BLOCK), lambda i: (0, i)),
            pl.BlockSpec((D, H), lambda i: (0, 0)),
            pl.BlockSpec((1, H), lambda i: (0, 0)),
            pl.BlockSpec((H, 1), lambda i: (0, 0)),
            pl.BlockSpec((1,), lambda i: (0,)),
        ],
        out_specs=(pl.BlockSpec((_EDGE_BLOCK,), lambda i: (i,)),
                   pl.BlockSpec((_EDGE_BLOCK,), lambda i: (i,))),
        out_shape=(jax.ShapeDtypeStruct((E,), jnp.float32),
                   jax.ShapeDtypeStruct((E,), jnp.int32)),
    )(edge_features, edge_index, W_edge, b_edge.reshape(1, H), w3c, b_pred)

    combine = functools.partial(
        pl.kernel,
        out_type=jax.ShapeDtypeStruct((E,), jnp.float32),
        mesh=plsc.VectorSubcoreMesh(core_axis_name="c", subcore_axis_name="s"),
        compiler_params=pltpu.CompilerParams(needs_layout_passes=False),
        scratch_types=[
            pltpu.VMEM((N,), jnp.float32),
            pltpu.VMEM((N,), jnp.float32),
            pltpu.VMEM((_EPW,), jnp.int32),
            pltpu.VMEM((_EPW,), jnp.float32),
            pltpu.VMEM((_EPW,), jnp.float32),
            pltpu.SemaphoreType.DMA,
        ],
    )(_combine_body)

    out = combine(p1, p2, comb, pe)
    return out.reshape(E, 1)


# speedup vs baseline: 2.8432x; 1.0039x over previous
import functools

import jax
import jax.numpy as jnp
from jax import lax
from jax.experimental import pallas as pl
from jax.experimental.pallas import tpu as pltpu
from jax.experimental.pallas import tpu_sc as plsc

N = 10000
E = 320000
D = 128
H = 128

_EDGE_BLOCK = 32768

_NUM_WORKERS = 32
_EPW = E // _NUM_WORKERS
_LANES = 16


def _node_proj_body(x_ref, w_ref, b_ref, wp_ref, o1_ref, o2_ref):
    x = x_ref[...].astype(jnp.bfloat16)
    w = w_ref[...].astype(jnp.bfloat16)
    n = jnp.dot(x, w, preferred_element_type=jnp.float32)
    n = jnp.maximum(n + b_ref[...], 0.0)
    nt = n.T
    o1_ref[...] = jnp.sum(nt * wp_ref[0:H], axis=0)
    o2_ref[...] = jnp.sum(nt * wp_ref[H:2 * H], axis=0)


def _edge_proj_body(x_ref, ei_ref, w_ref, b_ref, wp_ref, bp_ref,
                    o_ref, comb_ref):
    x = x_ref[...].astype(jnp.bfloat16)
    w = w_ref[...].astype(jnp.bfloat16)
    e = jnp.dot(x, w, preferred_element_type=jnp.float32)
    e = jnp.maximum(e + b_ref[...], 0.0)
    o_ref[...] = jnp.sum(e.T * wp_ref[2 * H:3 * H], axis=0) + bp_ref[...]
    comb_ref[...] = ei_ref[0] | (ei_ref[1] << 16)


def _combine_body(p1_hbm, p2_hbm, comb_hbm, pe_hbm, out_hbm,
                  tab1_v, tab2_v, comb_v, pe_v, out_v, sem):
    wid = lax.axis_index("s") * 2 + lax.axis_index("c")
    base = wid * _EPW
    sl = pl.ds(base, _EPW)
    copies = [
        pltpu.async_copy(p1_hbm, tab1_v, sem),
        pltpu.async_copy(p2_hbm, tab2_v, sem),
        pltpu.async_copy(comb_hbm.at[sl], comb_v, sem),
        pltpu.async_copy(pe_hbm.at[sl], pe_v, sem),
    ]
    for c in copies:
        c.wait()

    _UNROLL = 5

    def body(i, carry):
        for j in range(_UNROLL):
            o = (i * _UNROLL + j) * _LANES
            c = comb_v[pl.ds(o, _LANES)]
            s = c & 0xFFFF
            d = lax.shift_right_logical(c, 16)
            a = plsc.load_gather(tab1_v, [s])
            b = plsc.load_gather(tab2_v, [d])
            out_v[pl.ds(o, _LANES)] = a + b + pe_v[pl.ds(o, _LANES)]
        return carry

    lax.fori_loop(0, _EPW // (_LANES * _UNROLL), body, 0)
    pltpu.sync_copy(out_v, out_hbm.at[sl])


def kernel(node_features, edge_features, edge_index, W_node, b_node,
           W_edge, b_edge, W_pred, b_pred):
    _NODE_BLOCK = 5120
    p1, p2 = pl.pallas_call(
        _node_proj_body,
        grid=(pl.cdiv(N, _NODE_BLOCK),),
        in_specs=[
            pl.BlockSpec((_NODE_BLOCK, D), lambda i: (i, 0)),
            pl.BlockSpec((D, H), lambda i: (0, 0)),
            pl.BlockSpec((1, H), lambda i: (0, 0)),
            pl.BlockSpec((3 * H, 1), lambda i: (0, 0)),
        ],
        out_specs=(pl.BlockSpec((_NODE_BLOCK,), lambda i: (i,)),
                   pl.BlockSpec((_NODE_BLOCK,), lambda i: (i,))),
        out_shape=(jax.ShapeDtypeStruct((N,), jnp.float32),
                   jax.ShapeDtypeStruct((N,), jnp.float32)),
    )(node_features, W_node, b_node.reshape(1, H), W_pred)

    pe, comb = pl.pallas_call(
        _edge_proj_body,
        grid=(pl.cdiv(E, _EDGE_BLOCK),),
        in_specs=[
            pl.BlockSpec((_EDGE_BLOCK, D), lambda i: (i, 0)),
            pl.BlockSpec((2, _EDGE_BLOCK), lambda i: (0, i)),
            pl.BlockSpec((D, H), lambda i: (0, 0)),
            pl.BlockSpec((1, H), lambda i: (0, 0)),
            pl.BlockSpec((3 * H, 1), lambda i: (0, 0)),
            pl.BlockSpec((1,), lambda i: (0,)),
        ],
        out_specs=(pl.BlockSpec((_EDGE_BLOCK,), lambda i: (i,)),
                   pl.BlockSpec((_EDGE_BLOCK,), lambda i: (i,))),
        out_shape=(jax.ShapeDtypeStruct((E,), jnp.float32),
                   jax.ShapeDtypeStruct((E,), jnp.int32)),
    )(edge_features, edge_index, W_edge, b_edge.reshape(1, H), W_pred, b_pred)

    combine = functools.partial(
        pl.kernel,
        out_type=jax.ShapeDtypeStruct((E,), jnp.float32),
        mesh=plsc.VectorSubcoreMesh(core_axis_name="c", subcore_axis_name="s"),
        compiler_params=pltpu.CompilerParams(needs_layout_passes=False),
        scratch_types=[
            pltpu.VMEM((N,), jnp.float32),
            pltpu.VMEM((N,), jnp.float32),
            pltpu.VMEM((_EPW,), jnp.int32),
            pltpu.VMEM((_EPW,), jnp.float32),
            pltpu.VMEM((_EPW,), jnp.float32),
            pltpu.SemaphoreType.DMA,
        ],
    )(_combine_body)

    out = combine(p1, p2, comb, pe)
    return out.reshape(E, 1)
